# baseline (speedup 1.0000x reference)
import functools

import jax
import jax.numpy as jnp
from jax import lax
from jax.experimental import pallas as pl
from jax.experimental.pallas import tpu as pltpu
from jax.experimental.pallas import tpu_sc as plsc

N = 10000
E = 320000
D = 128
BN_EPS = 1e-5

NC = 2
NS = 16
NW = NC * NS
EPW = E // NW
CH = 40
NCHUNK = EPW // CH
R0 = (N // NS) // 8 * 8
RTAIL = N - NS * R0

_sc_mesh = plsc.VectorSubcoreMesh(
    core_axis_name="c", subcore_axis_name="s", num_cores=NC, num_subcores=NS
)


def _deg_body(dst_hbm, zeros_hbm, out_hbm, idx_v, ones_v, sdeg, dsem):
    c = lax.axis_index("c")
    s = lax.axis_index("s")
    w = c * NS + s
    pltpu.sync_copy(dst_hbm.at[w], idx_v)

    for off in list(range(0, CH - 15, 16)) + [CH - 16]:
        ones_v[pl.ds(off, 16)] = jnp.ones((16,), jnp.float32)

    @pl.when(s == 0)
    def _zero():
        pltpu.sync_copy(zeros_hbm, sdeg)

    plsc.subcore_barrier()

    def acc(j, carry):
        pltpu.async_copy(ones_v, sdeg.at[idx_v.at[j]], dsem, add=True)
        return carry

    lax.fori_loop(0, NCHUNK, acc, 0)

    def drain(j, carry):
        pltpu.make_async_copy(ones_v, sdeg.at[idx_v.at[0]], dsem).wait()
        return carry

    lax.fori_loop(0, NCHUNK, drain, 0)
    plsc.subcore_barrier()

    @pl.when(s == 0)
    def _out():
        pltpu.sync_copy(sdeg, out_hbm.at[c])


_deg_call = pl.kernel(
    _deg_body,
    out_type=jax.ShapeDtypeStruct((NC, N), jnp.float32),
    mesh=_sc_mesh,
    scratch_types=[
        pltpu.VMEM((NCHUNK, CH), jnp.int32),
        pltpu.VMEM((CH,), jnp.float32),
        pltpu.VMEM_SHARED((N,), jnp.float32),
        pltpu.SemaphoreType.DMA,
    ],
)


NSLOT = 6


def _agg_body(hp_hbm, pidx_hbm, out_hbm, pidx, si, di, bufs, acc_sh,
              gs0, gs1, gs2, gs3, gs4, gs5, ss0, ss1, ss2, ss3, ss4, ss5):
    c = lax.axis_index("c")
    s = lax.axis_index("s")
    w = c * NS + s
    gsems = (gs0, gs1, gs2, gs3, gs4, gs5)
    ssems = (ss0, ss1, ss2, ss3, ss4, ss5)
    pltpu.sync_copy(pidx_hbm.at[pl.ds(pl.multiple_of(w * EPW, 8), EPW)], pidx)
    base = pl.multiple_of(s * R0, 8)
    myhp = hp_hbm.at[c]
    pltpu.sync_copy(myhp.at[pl.ds(base, R0)], acc_sh.at[pl.ds(base, R0)])

    @pl.when(s == NS - 1)
    def _init_tail():
        pltpu.sync_copy(myhp.at[pl.ds(NS * R0, RTAIL)], acc_sh.at[pl.ds(NS * R0, RTAIL)])

    plsc.subcore_barrier()

    def unpack(j, r):
        for off in list(range(0, CH - 15, 16)) + ([CH - 16] if CH % 16 else []):
            p = pidx[pl.ds(j * CH + off, 16)]
            si[r, pl.ds(off, 16)] = lax.bitwise_and(p, 0x3FFF)
            di[r, pl.ds(off, 16)] = lax.shift_right_logical(p, 14)

    def fire_gather(r):
        pltpu.async_copy(myhp.at[si.at[r]], bufs.at[r], gsems[r])

    def wait_gather(r):
        pltpu.make_async_copy(myhp.at[si.at[r]], bufs.at[r], gsems[r]).wait()

    def fire_scatter(r):
        pltpu.async_copy(bufs.at[r], acc_sh.at[di.at[r]], ssems[r], add=True)

    def wait_scatter(r):
        pltpu.make_async_copy(bufs.at[r], acc_sh.at[di.at[r]], ssems[r]).wait()

    for r in range(NSLOT):
        unpack(r, r)
        fire_gather(r)

    def sstep(i, carry):
        j0 = NSLOT * i
        for r in range(NSLOT):
            wait_gather(r)
            fire_scatter(r)
        for r in range(NSLOT):
            wait_scatter(r)

            @pl.when(j0 + r + NSLOT < NCHUNK)
            def _refill():
                unpack(j0 + r + NSLOT, r)
                fire_gather(r)

        return carry

    lax.fori_loop(0, NCHUNK // NSLOT, sstep, 0)
    for r in range(NCHUNK % NSLOT):
        wait_gather(r)
        fire_scatter(r)
    for r in range(NCHUNK % NSLOT):
        wait_scatter(r)
    plsc.subcore_barrier()
    pltpu.sync_copy(acc_sh.at[pl.ds(base, R0)], out_hbm.at[c].at[pl.ds(base, R0)])

    @pl.when(s == NS - 1)
    def _out_tail():
        pltpu.sync_copy(
            acc_sh.at[pl.ds(NS * R0, RTAIL)], out_hbm.at[c].at[pl.ds(NS * R0, RTAIL)]
        )


_agg_call = pl.kernel(
    _agg_body,
    out_type=jax.ShapeDtypeStruct((NC, N, D), jnp.float32),
    mesh=_sc_mesh,
    scratch_types=[
        pltpu.VMEM((EPW,), jnp.int32),
        pltpu.VMEM((NSLOT, CH), jnp.int32),
        pltpu.VMEM((NSLOT, CH), jnp.int32),
        pltpu.VMEM((NSLOT, CH, D), jnp.float32),
        pltpu.VMEM_SHARED((N, D), jnp.float32),
        pltpu.SemaphoreType.DMA,
        pltpu.SemaphoreType.DMA,
        pltpu.SemaphoreType.DMA,
        pltpu.SemaphoreType.DMA,
        pltpu.SemaphoreType.DMA,
        pltpu.SemaphoreType.DMA,
        pltpu.SemaphoreType.DMA,
        pltpu.SemaphoreType.DMA,
        pltpu.SemaphoreType.DMA,
        pltpu.SemaphoreType.DMA,
        pltpu.SemaphoreType.DMA,
        pltpu.SemaphoreType.DMA,
    ],
)


def _dinv_col(degp):
    deg = lax.dot_general(
        degp, jnp.ones((NC, 1), jnp.float32),
        (((0,), (0,)), ((), ())), preferred_element_type=jnp.float32,
    ) + 1.0
    return lax.rsqrt(jnp.maximum(deg, 1.0))


def _mm_body(x_ref, w_ref, degp_ref, hp_ref):
    dinv = _dinv_col(degp_ref[...])
    h = jnp.dot(x_ref[...], w_ref[...], preferred_element_type=jnp.float32)
    hp = h * dinv
    hp_ref[0] = hp
    hp_ref[1] = hp


_mm_call = pl.pallas_call(
    _mm_body,
    out_shape=jax.ShapeDtypeStruct((NC, N, D), jnp.float32),
)


def _bn_body(s_ref, hp_ref, degp_ref, b_ref, g_ref, be_ref, z_ref):
    dinv = _dinv_col(degp_ref[...])
    t = (s_ref[0] + s_ref[1] - hp_ref[0]) * dinv + b_ref[...]
    r = jnp.maximum(t, 0.0)
    mean = jnp.mean(r, axis=0, keepdims=True)
    cent = r - mean
    var = jnp.mean(cent * cent, axis=0, keepdims=True)
    z_ref[...] = cent * lax.rsqrt(var + BN_EPS) * g_ref[...] + be_ref[...]


_bn_call = pl.pallas_call(
    _bn_body,
    out_shape=jax.ShapeDtypeStruct((N, D), jnp.float32),
)


@jax.jit
def kernel(x, edge_index, W, b, gamma, beta):
    src = edge_index[0]
    dst = edge_index[1]
    dst_r = dst.reshape(NW, NCHUNK, CH)
    packed = jnp.bitwise_or(jnp.left_shift(dst, 14), src)
    degp = _deg_call(dst_r, jnp.zeros((N,), jnp.float32))
    hp = _mm_call(x, W, degp)
    s_parts = _agg_call(hp, packed)
    z = _bn_call(
        s_parts, hp, degp,
        b.reshape(1, D), gamma.reshape(1, D), beta.reshape(1, D),
    )
    return z


# speedup vs baseline: 1.0213x; 1.0213x over previous
import functools

import jax
import jax.numpy as jnp
from jax import lax
from jax.experimental import pallas as pl
from jax.experimental.pallas import tpu as pltpu
from jax.experimental.pallas import tpu_sc as plsc

N = 10000
E = 320000
D = 128
BN_EPS = 1e-5

NC = 2
NS = 16
NW = NC * NS
EPW = E // NW
CH = 40
NCHUNK = EPW // CH
R0 = (N // NS) // 8 * 8
RTAIL = N - NS * R0

_sc_mesh = plsc.VectorSubcoreMesh(
    core_axis_name="c", subcore_axis_name="s", num_cores=NC, num_subcores=NS
)


def _deg_body(dst_hbm, zeros_hbm, out_hbm, idx_v, ones_v, sdeg, dsem):
    c = lax.axis_index("c")
    s = lax.axis_index("s")
    w = c * NS + s
    pltpu.sync_copy(dst_hbm.at[w], idx_v)

    for off in list(range(0, CH - 15, 16)) + [CH - 16]:
        ones_v[pl.ds(off, 16)] = jnp.ones((16,), jnp.float32)

    @pl.when(s == 0)
    def _zero():
        pltpu.sync_copy(zeros_hbm, sdeg)

    plsc.subcore_barrier()

    def acc(j, carry):
        pltpu.async_copy(ones_v, sdeg.at[idx_v.at[j]], dsem, add=True)
        return carry

    lax.fori_loop(0, NCHUNK, acc, 0)

    def drain(j, carry):
        pltpu.make_async_copy(ones_v, sdeg.at[idx_v.at[0]], dsem).wait()
        return carry

    lax.fori_loop(0, NCHUNK, drain, 0)
    plsc.subcore_barrier()

    @pl.when(s == 0)
    def _out():
        pltpu.sync_copy(sdeg, out_hbm.at[c])


_deg_call = pl.kernel(
    _deg_body,
    out_type=jax.ShapeDtypeStruct((NC, N), jnp.float32),
    mesh=_sc_mesh,
    scratch_types=[
        pltpu.VMEM((NCHUNK, CH), jnp.int32),
        pltpu.VMEM((CH,), jnp.float32),
        pltpu.VMEM_SHARED((N,), jnp.float32),
        pltpu.SemaphoreType.DMA,
    ],
)


NSLOT = 6


def _agg_body(hp_hbm, pidx_hbm, out_hbm, pidx, si, di, bufs, acc_sh,
              gs0, gs1, gs2, gs3, gs4, gs5, ss0, ss1, ss2, ss3, ss4, ss5):
    c = lax.axis_index("c")
    s = lax.axis_index("s")
    w = c * NS + s
    gsems = (gs0, gs1, gs2, gs3, gs4, gs5)
    ssems = (ss0, ss1, ss2, ss3, ss4, ss5)
    pltpu.sync_copy(pidx_hbm.at[pl.ds(pl.multiple_of(w * EPW, 8), EPW)], pidx)
    base = pl.multiple_of(s * R0, 8)
    myhp = hp_hbm
    pltpu.sync_copy(myhp.at[pl.ds(base, R0)], acc_sh.at[pl.ds(base, R0)])

    @pl.when(s == NS - 1)
    def _init_tail():
        pltpu.sync_copy(myhp.at[pl.ds(NS * R0, RTAIL)], acc_sh.at[pl.ds(NS * R0, RTAIL)])

    plsc.subcore_barrier()

    def unpack(j, r):
        for off in list(range(0, CH - 15, 16)) + ([CH - 16] if CH % 16 else []):
            p = pidx[pl.ds(j * CH + off, 16)]
            si[r, pl.ds(off, 16)] = lax.bitwise_and(p, 0x3FFF)
            di[r, pl.ds(off, 16)] = lax.shift_right_logical(p, 14)

    def fire_gather(r):
        pltpu.async_copy(myhp.at[si.at[r]], bufs.at[r], gsems[r])

    def wait_gather(r):
        pltpu.make_async_copy(myhp.at[si.at[r]], bufs.at[r], gsems[r]).wait()

    def fire_scatter(r):
        pltpu.async_copy(bufs.at[r], acc_sh.at[di.at[r]], ssems[r], add=True)

    def wait_scatter(r):
        pltpu.make_async_copy(bufs.at[r], acc_sh.at[di.at[r]], ssems[r]).wait()

    for r in range(NSLOT):
        unpack(r, r)
        fire_gather(r)

    def sstep(i, carry):
        j0 = NSLOT * i
        for r in range(NSLOT):
            wait_gather(r)
            fire_scatter(r)
        for r in range(NSLOT):
            wait_scatter(r)

            @pl.when(j0 + r + NSLOT < NCHUNK)
            def _refill():
                unpack(j0 + r + NSLOT, r)
                fire_gather(r)

        return carry

    lax.fori_loop(0, NCHUNK // NSLOT, sstep, 0)
    for r in range(NCHUNK % NSLOT):
        wait_gather(r)
        fire_scatter(r)
    for r in range(NCHUNK % NSLOT):
        wait_scatter(r)
    plsc.subcore_barrier()
    pltpu.sync_copy(acc_sh.at[pl.ds(base, R0)], out_hbm.at[c].at[pl.ds(base, R0)])

    @pl.when(s == NS - 1)
    def _out_tail():
        pltpu.sync_copy(
            acc_sh.at[pl.ds(NS * R0, RTAIL)], out_hbm.at[c].at[pl.ds(NS * R0, RTAIL)]
        )


_agg_call = pl.kernel(
    _agg_body,
    out_type=jax.ShapeDtypeStruct((NC, N, D), jnp.float32),
    mesh=_sc_mesh,
    scratch_types=[
        pltpu.VMEM((EPW,), jnp.int32),
        pltpu.VMEM((NSLOT, CH), jnp.int32),
        pltpu.VMEM((NSLOT, CH), jnp.int32),
        pltpu.VMEM((NSLOT, CH, D), jnp.float32),
        pltpu.VMEM_SHARED((N, D), jnp.float32),
        pltpu.SemaphoreType.DMA,
        pltpu.SemaphoreType.DMA,
        pltpu.SemaphoreType.DMA,
        pltpu.SemaphoreType.DMA,
        pltpu.SemaphoreType.DMA,
        pltpu.SemaphoreType.DMA,
        pltpu.SemaphoreType.DMA,
        pltpu.SemaphoreType.DMA,
        pltpu.SemaphoreType.DMA,
        pltpu.SemaphoreType.DMA,
        pltpu.SemaphoreType.DMA,
        pltpu.SemaphoreType.DMA,
    ],
)


def _dinv_col(degp):
    deg = lax.dot_general(
        degp, jnp.ones((NC, 1), jnp.float32),
        (((0,), (0,)), ((), ())), preferred_element_type=jnp.float32,
    ) + 1.0
    return lax.rsqrt(jnp.maximum(deg, 1.0))


def _mm_body(x_ref, w_ref, degp_ref, ei_ref, hp_ref, pk_ref):
    dinv = _dinv_col(degp_ref[...])
    h = jnp.dot(x_ref[...], w_ref[...], preferred_element_type=jnp.float32)
    hp_ref[...] = h * dinv
    pk_ref[...] = lax.bitwise_or(lax.shift_left(ei_ref[1], 14), ei_ref[0])


_mm_call = pl.pallas_call(
    _mm_body,
    out_shape=[
        jax.ShapeDtypeStruct((N, D), jnp.float32),
        jax.ShapeDtypeStruct((E,), jnp.int32),
    ],
)


def _bn_body(s_ref, hp_ref, degp_ref, b_ref, g_ref, be_ref, z_ref):
    dinv = _dinv_col(degp_ref[...])
    t = (s_ref[0] + s_ref[1] - hp_ref[...]) * dinv + b_ref[...]
    r = jnp.maximum(t, 0.0)
    mean = jnp.mean(r, axis=0, keepdims=True)
    cent = r - mean
    var = jnp.mean(cent * cent, axis=0, keepdims=True)
    z_ref[...] = cent * lax.rsqrt(var + BN_EPS) * g_ref[...] + be_ref[...]


_bn_call = pl.pallas_call(
    _bn_body,
    out_shape=jax.ShapeDtypeStruct((N, D), jnp.float32),
)


@jax.jit
def kernel(x, edge_index, W, b, gamma, beta):
    dst_r = edge_index[1].reshape(NW, NCHUNK, CH)
    degp = _deg_call(dst_r, jnp.zeros((N,), jnp.float32))
    hp, packed = _mm_call(x, W, degp, edge_index)
    s_parts = _agg_call(hp, packed)
    z = _bn_call(
        s_parts, hp, degp,
        b.reshape(1, D), gamma.reshape(1, D), beta.reshape(1, D),
    )
    return z


# speedup vs baseline: 1.0905x; 1.0678x over previous
import functools

import jax
import jax.numpy as jnp
from jax import lax
from jax.experimental import pallas as pl
from jax.experimental.pallas import tpu as pltpu
from jax.experimental.pallas import tpu_sc as plsc

N = 10000
E = 320000
D = 128
BN_EPS = 1e-5

NC = 2
NS = 16
NW = NC * NS
EPW = E // NW
CH = 40
NCHUNK = EPW // CH
R0 = (N // NS) // 8 * 8
RTAIL = N - NS * R0

_sc_mesh = plsc.VectorSubcoreMesh(
    core_axis_name="c", subcore_axis_name="s", num_cores=NC, num_subcores=NS
)


def _deg_body(pk_hbm, zeros_hbm, out_hbm, pkf, idx_v, ones_v, sdeg, dsem):
    c = lax.axis_index("c")
    s = lax.axis_index("s")
    w = c * NS + s
    pltpu.sync_copy(pk_hbm.at[pl.ds(pl.multiple_of(w * EPW, 8), EPW)], pkf)

    _offs = list(range(0, CH - 15, 16)) + ([CH - 16] if CH % 16 else [])

    def stage(j, carry):
        for off in _offs:
            idx_v[j, pl.ds(off, 16)] = lax.shift_right_logical(
                pkf[pl.ds(j * CH + off, 16)], 14)
        return carry

    lax.fori_loop(0, NCHUNK, stage, 0)

    for off in _offs:
        ones_v[pl.ds(off, 16)] = jnp.ones((16,), jnp.float32)

    @pl.when(s == 0)
    def _zero():
        pltpu.sync_copy(zeros_hbm, sdeg)

    plsc.subcore_barrier()

    def acc(j, carry):
        pltpu.async_copy(ones_v, sdeg.at[idx_v.at[j]], dsem, add=True)
        return carry

    lax.fori_loop(0, NCHUNK, acc, 0)

    def drain(j, carry):
        pltpu.make_async_copy(ones_v, sdeg.at[idx_v.at[0]], dsem).wait()
        return carry

    lax.fori_loop(0, NCHUNK, drain, 0)
    plsc.subcore_barrier()

    @pl.when(s == 0)
    def _out():
        pltpu.sync_copy(sdeg, out_hbm.at[c])


_deg_call = pl.kernel(
    _deg_body,
    out_type=jax.ShapeDtypeStruct((NC, N), jnp.float32),
    mesh=_sc_mesh,
    scratch_types=[
        pltpu.VMEM((EPW,), jnp.int32),
        pltpu.VMEM((NCHUNK, CH), jnp.int32),
        pltpu.VMEM((CH,), jnp.float32),
        pltpu.VMEM_SHARED((N,), jnp.float32),
        pltpu.SemaphoreType.DMA,
    ],
)


NSLOT = 6


def _agg_body(hp_hbm, pidx_hbm, out_hbm, pidx, si, di, bufs, acc_sh,
              gs0, gs1, gs2, gs3, gs4, gs5, ss0, ss1, ss2, ss3, ss4, ss5):
    c = lax.axis_index("c")
    s = lax.axis_index("s")
    w = c * NS + s
    gsems = (gs0, gs1, gs2, gs3, gs4, gs5)
    ssems = (ss0, ss1, ss2, ss3, ss4, ss5)
    pltpu.sync_copy(pidx_hbm.at[pl.ds(pl.multiple_of(w * EPW, 8), EPW)], pidx)
    base = pl.multiple_of(s * R0, 8)
    myhp = hp_hbm
    pltpu.sync_copy(myhp.at[pl.ds(base, R0)], acc_sh.at[pl.ds(base, R0)])

    @pl.when(s == NS - 1)
    def _init_tail():
        pltpu.sync_copy(myhp.at[pl.ds(NS * R0, RTAIL)], acc_sh.at[pl.ds(NS * R0, RTAIL)])

    plsc.subcore_barrier()

    def unpack(j, r):
        for off in list(range(0, CH - 15, 16)) + ([CH - 16] if CH % 16 else []):
            p = pidx[pl.ds(j * CH + off, 16)]
            si[r, pl.ds(off, 16)] = lax.bitwise_and(p, 0x3FFF)
            di[r, pl.ds(off, 16)] = lax.shift_right_logical(p, 14)

    def fire_gather(r):
        pltpu.async_copy(myhp.at[si.at[r]], bufs.at[r], gsems[r])

    def wait_gather(r):
        pltpu.make_async_copy(myhp.at[si.at[r]], bufs.at[r], gsems[r]).wait()

    def fire_scatter(r):
        pltpu.async_copy(bufs.at[r], acc_sh.at[di.at[r]], ssems[r], add=True)

    def wait_scatter(r):
        pltpu.make_async_copy(bufs.at[r], acc_sh.at[di.at[r]], ssems[r]).wait()

    for r in range(NSLOT):
        unpack(r, r)
        fire_gather(r)

    def sstep(i, carry):
        j0 = NSLOT * i
        for r in range(NSLOT):
            wait_gather(r)
            fire_scatter(r)
        for r in range(NSLOT):
            wait_scatter(r)

            @pl.when(j0 + r + NSLOT < NCHUNK)
            def _refill():
                unpack(j0 + r + NSLOT, r)
                fire_gather(r)

        return carry

    lax.fori_loop(0, NCHUNK // NSLOT, sstep, 0)
    for r in range(NCHUNK % NSLOT):
        wait_gather(r)
        fire_scatter(r)
    for r in range(NCHUNK % NSLOT):
        wait_scatter(r)
    plsc.subcore_barrier()
    pltpu.sync_copy(acc_sh.at[pl.ds(base, R0)], out_hbm.at[c].at[pl.ds(base, R0)])

    @pl.when(s == NS - 1)
    def _out_tail():
        pltpu.sync_copy(
            acc_sh.at[pl.ds(NS * R0, RTAIL)], out_hbm.at[c].at[pl.ds(NS * R0, RTAIL)]
        )


_agg_call = pl.kernel(
    _agg_body,
    out_type=jax.ShapeDtypeStruct((NC, N, D), jnp.float32),
    mesh=_sc_mesh,
    scratch_types=[
        pltpu.VMEM((EPW,), jnp.int32),
        pltpu.VMEM((NSLOT, CH), jnp.int32),
        pltpu.VMEM((NSLOT, CH), jnp.int32),
        pltpu.VMEM((NSLOT, CH, D), jnp.float32),
        pltpu.VMEM_SHARED((N, D), jnp.float32),
        pltpu.SemaphoreType.DMA,
        pltpu.SemaphoreType.DMA,
        pltpu.SemaphoreType.DMA,
        pltpu.SemaphoreType.DMA,
        pltpu.SemaphoreType.DMA,
        pltpu.SemaphoreType.DMA,
        pltpu.SemaphoreType.DMA,
        pltpu.SemaphoreType.DMA,
        pltpu.SemaphoreType.DMA,
        pltpu.SemaphoreType.DMA,
        pltpu.SemaphoreType.DMA,
        pltpu.SemaphoreType.DMA,
    ],
)


def _dinv_col(degp):
    deg = lax.dot_general(
        degp, jnp.ones((NC, 1), jnp.float32),
        (((0,), (0,)), ((), ())), preferred_element_type=jnp.float32,
    ) + 1.0
    return lax.rsqrt(jnp.maximum(deg, 1.0))


def _prep_body(x_ref, w_ref, ei_ref, h_ref, pk_ref):
    h_ref[...] = jnp.dot(x_ref[...], w_ref[...], preferred_element_type=jnp.float32)
    pk_ref[...] = lax.bitwise_or(lax.shift_left(ei_ref[1], 14), ei_ref[0])


def _scale_body(h_ref, degp_ref, hp_ref):
    hp_ref[...] = h_ref[...] * _dinv_col(degp_ref[...])


_prep_call = pl.pallas_call(
    _prep_body,
    out_shape=[
        jax.ShapeDtypeStruct((N, D), jnp.float32),
        jax.ShapeDtypeStruct((E,), jnp.int32),
    ],
)

_scale_call = pl.pallas_call(
    _scale_body,
    out_shape=jax.ShapeDtypeStruct((N, D), jnp.float32),
)


def _bn_body(s_ref, hp_ref, degp_ref, b_ref, g_ref, be_ref, z_ref):
    dinv = _dinv_col(degp_ref[...])
    t = (s_ref[0] + s_ref[1] - hp_ref[...]) * dinv + b_ref[...]
    r = jnp.maximum(t, 0.0)
    mean = jnp.mean(r, axis=0, keepdims=True)
    cent = r - mean
    var = jnp.mean(cent * cent, axis=0, keepdims=True)
    z_ref[...] = cent * lax.rsqrt(var + BN_EPS) * g_ref[...] + be_ref[...]


_bn_call = pl.pallas_call(
    _bn_body,
    out_shape=jax.ShapeDtypeStruct((N, D), jnp.float32),
)


@jax.jit
def kernel(x, edge_index, W, b, gamma, beta):
    h_raw, packed = _prep_call(x, W, edge_index)
    degp = _deg_call(packed, jnp.zeros((N,), jnp.float32))
    hp = _scale_call(h_raw, degp)
    s_parts = _agg_call(hp, packed)
    z = _bn_call(
        s_parts, hp, degp,
        b.reshape(1, D), gamma.reshape(1, D), beta.reshape(1, D),
    )
    return z
